# Initial kernel scaffold; baseline (speedup 1.0000x reference)
import jax
import jax.numpy as jnp
from jax.experimental import pallas as pl


def kernel(tokens, chain_ids, embed_table, ln_gamma, ln_beta):
    raise NotImplementedError("write your pallas kernel here")


# speedup vs baseline: 3.1254x; 3.1254x over previous
import jax
import jax.numpy as jnp
from jax.experimental import pallas as pl
from jax.experimental.pallas import tpu as pltpu

VOCAB = 33
EMBED_DIM = 1280
PADDING_IDX = 1
MASK_IDX = 32
LN_EPS = 1e-5
VPAD = 64

B = 32
S = 1024
TBLK = 512
NBLK = S // TBLK


def _stage_a_body(tokens_ref, table_ref, gamma_ref, beta_ref, n2_ref):
    tok = tokens_ref[0]
    n_nonpad = jnp.sum((tok != PADDING_IDX).astype(jnp.float32))
    n_mask = jnp.sum((tok == MASK_IDX).astype(jnp.float32))
    s = 0.88 * n_nonpad / (n_nonpad - n_mask)

    tab = table_ref[...]
    rid = jax.lax.broadcasted_iota(jnp.int32, (VPAD, EMBED_DIM), 0)
    keep = ((rid != PADDING_IDX) & (rid != MASK_IDX)).astype(jnp.float32)
    x = tab * keep * s
    mean = jnp.mean(x, axis=1, keepdims=True)
    var = jnp.mean((x - mean) * (x - mean), axis=1, keepdims=True)
    inv = jax.lax.rsqrt(var + LN_EPS)
    n = (x - mean) * inv * gamma_ref[...] + beta_ref[...]

    hi = n.astype(jnp.bfloat16)
    lo = (n - hi.astype(jnp.float32)).astype(jnp.bfloat16)
    n2_ref[...] = jnp.concatenate([hi, lo], axis=0)


def _stage_b_body(tcol_ref, n2_ref, out_ref):
    t = tcol_ref[0]
    v = jax.lax.broadcasted_iota(jnp.int32, (TBLK, 2 * VPAD), 1) & (VPAD - 1)
    onehot = (t == v).astype(jnp.bfloat16)
    out_ref[...] = jax.lax.dot_general(
        onehot, n2_ref[...],
        (((1,), (0,)), ((), ())),
        preferred_element_type=jnp.float32,
    )


def kernel(tokens, chain_ids, embed_table, ln_gamma, ln_beta):
    del chain_ids
    tokens = tokens.astype(jnp.int32)
    table_pad = jnp.zeros((VPAD, EMBED_DIM), jnp.float32).at[:VOCAB].set(embed_table)

    n2 = pl.pallas_call(
        _stage_a_body,
        grid=(B,),
        in_specs=[
            pl.BlockSpec((1, 1, S), lambda b: (b, 0, 0)),
            pl.BlockSpec((VPAD, EMBED_DIM), lambda b: (0, 0)),
            pl.BlockSpec((1, EMBED_DIM), lambda b: (0, 0)),
            pl.BlockSpec((1, EMBED_DIM), lambda b: (0, 0)),
        ],
        out_specs=pl.BlockSpec((2 * VPAD, EMBED_DIM), lambda b: (b, 0)),
        out_shape=jax.ShapeDtypeStruct((B * 2 * VPAD, EMBED_DIM), jnp.bfloat16),
    )(
        tokens.reshape(B, 1, S),
        table_pad,
        ln_gamma.reshape(1, EMBED_DIM),
        ln_beta.reshape(1, EMBED_DIM),
    )

    out = pl.pallas_call(
        _stage_b_body,
        grid=(B, NBLK),
        in_specs=[
            pl.BlockSpec((1, TBLK, 1), lambda b, j: (b * NBLK + j, 0, 0)),
            pl.BlockSpec((2 * VPAD, EMBED_DIM), lambda b, j: (b, 0)),
        ],
        out_specs=pl.BlockSpec((TBLK, EMBED_DIM), lambda b, j: (b * NBLK + j, 0)),
        out_shape=jax.ShapeDtypeStruct((B * S, EMBED_DIM), jnp.float32),
    )(
        tokens.reshape(B * NBLK, TBLK, 1),
        n2,
    )
    return out.reshape(B, S, EMBED_DIM)


# speedup vs baseline: 4.4136x; 1.4122x over previous
import jax
import jax.numpy as jnp
from jax.experimental import pallas as pl
from jax.experimental.pallas import tpu as pltpu

VOCAB = 33
EMBED_DIM = 1280
PADDING_IDX = 1
MASK_IDX = 32
LN_EPS = 1e-5
VPAD = 64

B = 32
S = 1024
TBLK = 512
NBLK = S // TBLK


def _stage_a_body(tokens_ref, table_ref, gamma_ref, beta_ref, n2_ref):
    tok = tokens_ref[...]
    n_nonpad = jnp.sum((tok != PADDING_IDX).astype(jnp.float32), axis=1, keepdims=True)
    n_mask = jnp.sum((tok == MASK_IDX).astype(jnp.float32), axis=1, keepdims=True)
    s = 0.88 * n_nonpad / (n_nonpad - n_mask)

    tab = table_ref[...]
    rid = jax.lax.broadcasted_iota(jnp.int32, (VPAD, EMBED_DIM), 0)
    keep = ((rid != PADDING_IDX) & (rid != MASK_IDX)).astype(jnp.float32)
    tabk = (tab * keep)[None]
    x = tabk * s[:, :, None]
    mean = jnp.mean(x, axis=2, keepdims=True)
    var = jnp.mean((x - mean) * (x - mean), axis=2, keepdims=True)
    inv = jax.lax.rsqrt(var + LN_EPS)
    n = (x - mean) * inv * gamma_ref[...][None] + beta_ref[...][None]

    hi = n.astype(jnp.bfloat16)
    lo = (n - hi.astype(jnp.float32)).astype(jnp.bfloat16)
    n2_ref[...] = jnp.concatenate([hi, lo], axis=1)


def _stage_b_body(trow_ref, n2_ref, out_ref):
    t = trow_ref[0]
    v = jax.lax.broadcasted_iota(jnp.int32, (2 * VPAD, TBLK), 0) & (VPAD - 1)
    onehot_t = (t == v).astype(jnp.bfloat16)
    out_ref[...] = jax.lax.dot_general(
        onehot_t, n2_ref[0],
        (((0,), (0,)), ((), ())),
        preferred_element_type=jnp.float32,
    )


def kernel(tokens, chain_ids, embed_table, ln_gamma, ln_beta):
    del chain_ids
    tokens = tokens.astype(jnp.int32)
    table_pad = jnp.zeros((VPAD, EMBED_DIM), jnp.float32).at[:VOCAB].set(embed_table)

    n2 = pl.pallas_call(
        _stage_a_body,
        grid=(1,),
        in_specs=[
            pl.BlockSpec((B, S), lambda i: (0, 0)),
            pl.BlockSpec((VPAD, EMBED_DIM), lambda i: (0, 0)),
            pl.BlockSpec((1, EMBED_DIM), lambda i: (0, 0)),
            pl.BlockSpec((1, EMBED_DIM), lambda i: (0, 0)),
        ],
        out_specs=pl.BlockSpec((B, 2 * VPAD, EMBED_DIM), lambda i: (0, 0, 0)),
        out_shape=jax.ShapeDtypeStruct((B, 2 * VPAD, EMBED_DIM), jnp.bfloat16),
    )(
        tokens,
        table_pad,
        ln_gamma.reshape(1, EMBED_DIM),
        ln_beta.reshape(1, EMBED_DIM),
    )

    out = pl.pallas_call(
        _stage_b_body,
        grid=(B, NBLK),
        in_specs=[
            pl.BlockSpec((1, 1, TBLK), lambda b, j: (b * NBLK + j, 0, 0)),
            pl.BlockSpec((1, 2 * VPAD, EMBED_DIM), lambda b, j: (b, 0, 0)),
        ],
        out_specs=pl.BlockSpec((TBLK, EMBED_DIM), lambda b, j: (b * NBLK + j, 0)),
        out_shape=jax.ShapeDtypeStruct((B * S, EMBED_DIM), jnp.float32),
    )(
        tokens.reshape(B * NBLK, 1, TBLK),
        n2,
    )
    return out.reshape(B, S, EMBED_DIM)


# speedup vs baseline: 5.5135x; 1.2492x over previous
import jax
import jax.numpy as jnp
from jax.experimental import pallas as pl
from jax.experimental.pallas import tpu as pltpu

VOCAB = 33
EMBED_DIM = 1280
PADDING_IDX = 1
MASK_IDX = 32
LN_EPS = 1e-5
VPAD = 64

B = 32
S = 1024
TBLK = 1024
NBLK = S // TBLK


def _stage_a_body(tokens_ref, table_ref, gamma_ref, beta_ref, n2_ref):
    tok = tokens_ref[...]
    n_nonpad = jnp.sum((tok != PADDING_IDX).astype(jnp.float32), axis=1, keepdims=True)
    n_mask = jnp.sum((tok == MASK_IDX).astype(jnp.float32), axis=1, keepdims=True)
    s = 0.88 * n_nonpad / (n_nonpad - n_mask)

    tab = table_ref[...]
    rid = jax.lax.broadcasted_iota(jnp.int32, (VPAD, EMBED_DIM), 0)
    keep = ((rid != PADDING_IDX) & (rid != MASK_IDX)).astype(jnp.float32)
    tabk = (tab * keep)[None]
    x = tabk * s[:, :, None]
    mean = jnp.mean(x, axis=2, keepdims=True)
    var = jnp.mean((x - mean) * (x - mean), axis=2, keepdims=True)
    inv = jax.lax.rsqrt(var + LN_EPS)
    n = (x - mean) * inv * gamma_ref[...][None] + beta_ref[...][None]

    hi = n.astype(jnp.bfloat16)
    lo = (n - hi.astype(jnp.float32)).astype(jnp.bfloat16)
    n2_ref[...] = jnp.concatenate([hi, lo], axis=1)


def _stage_b_body(trow_ref, n2_ref, out_ref):
    t = trow_ref[0]
    v = jax.lax.broadcasted_iota(jnp.int32, (2 * VPAD, TBLK), 0) & (VPAD - 1)
    onehot_t = (t == v).astype(jnp.bfloat16)
    out_ref[...] = jax.lax.dot_general(
        onehot_t, n2_ref[0],
        (((0,), (0,)), ((), ())),
        preferred_element_type=jnp.float32,
    )


def kernel(tokens, chain_ids, embed_table, ln_gamma, ln_beta):
    del chain_ids
    tokens = tokens.astype(jnp.int32)
    table_pad = jnp.zeros((VPAD, EMBED_DIM), jnp.float32).at[:VOCAB].set(embed_table)

    n2 = pl.pallas_call(
        _stage_a_body,
        grid=(1,),
        in_specs=[
            pl.BlockSpec((B, S), lambda i: (0, 0)),
            pl.BlockSpec((VPAD, EMBED_DIM), lambda i: (0, 0)),
            pl.BlockSpec((1, EMBED_DIM), lambda i: (0, 0)),
            pl.BlockSpec((1, EMBED_DIM), lambda i: (0, 0)),
        ],
        out_specs=pl.BlockSpec((B, 2 * VPAD, EMBED_DIM), lambda i: (0, 0, 0)),
        out_shape=jax.ShapeDtypeStruct((B, 2 * VPAD, EMBED_DIM), jnp.bfloat16),
    )(
        tokens,
        table_pad,
        ln_gamma.reshape(1, EMBED_DIM),
        ln_beta.reshape(1, EMBED_DIM),
    )

    out = pl.pallas_call(
        _stage_b_body,
        grid=(B, NBLK),
        in_specs=[
            pl.BlockSpec((1, 1, TBLK), lambda b, j: (b * NBLK + j, 0, 0)),
            pl.BlockSpec((1, 2 * VPAD, EMBED_DIM), lambda b, j: (b, 0, 0)),
        ],
        out_specs=pl.BlockSpec((TBLK, EMBED_DIM), lambda b, j: (b * NBLK + j, 0)),
        out_shape=jax.ShapeDtypeStruct((B * S, EMBED_DIM), jnp.float32),
    )(
        tokens.reshape(B * NBLK, 1, TBLK),
        n2,
    )
    return out.reshape(B, S, EMBED_DIM)


# speedup vs baseline: 5.8327x; 1.0579x over previous
import jax
import jax.numpy as jnp
from jax.experimental import pallas as pl
from jax.experimental.pallas import tpu as pltpu

VOCAB = 33
EMBED_DIM = 1280
PADDING_IDX = 1
MASK_IDX = 32
LN_EPS = 1e-5
VPAD = 64

B = 32
S = 1024


def _stage_a_body(tokens_ref, table_ref, gamma_ref, beta_ref, n2_ref):
    tok = tokens_ref[...]
    n_nonpad = jnp.sum((tok != PADDING_IDX).astype(jnp.float32), axis=1, keepdims=True)
    n_mask = jnp.sum((tok == MASK_IDX).astype(jnp.float32), axis=1, keepdims=True)
    s = 0.88 * n_nonpad / (n_nonpad - n_mask)

    tab = table_ref[...]
    rid = jax.lax.broadcasted_iota(jnp.int32, (VPAD, EMBED_DIM), 0)
    keep = ((rid != PADDING_IDX) & (rid != MASK_IDX)).astype(jnp.float32)
    tabk = (tab * keep)[None]
    x = tabk * s[:, :, None]
    mean = jnp.mean(x, axis=2, keepdims=True)
    var = jnp.mean((x - mean) * (x - mean), axis=2, keepdims=True)
    inv = jax.lax.rsqrt(var + LN_EPS)
    n = (x - mean) * inv * gamma_ref[...][None] + beta_ref[...][None]

    hi = n.astype(jnp.bfloat16)
    lo = (n - hi.astype(jnp.float32)).astype(jnp.bfloat16)
    n2_ref[...] = jnp.concatenate([hi, lo], axis=1)


RPG = 2
TBLK = RPG * S
K = RPG * 2 * VPAD


def _stage_b_body(trow_ref, n2_ref, out_ref):
    t = trow_ref[0]
    v = jax.lax.broadcasted_iota(jnp.int32, (K, TBLK), 0)
    i = jax.lax.broadcasted_iota(jnp.int32, (K, TBLK), 1)
    onehot_t = ((t == (v & (VPAD - 1)))
                & ((v >> 7) == (i >> 10))).astype(jnp.bfloat16)
    out_ref[...] = jax.lax.dot_general(
        onehot_t, n2_ref[...],
        (((0,), (0,)), ((), ())),
        preferred_element_type=jnp.float32,
    )


def kernel(tokens, chain_ids, embed_table, ln_gamma, ln_beta):
    del chain_ids
    tokens = tokens.astype(jnp.int32)
    table_pad = jnp.zeros((VPAD, EMBED_DIM), jnp.float32).at[:VOCAB].set(embed_table)

    n2 = pl.pallas_call(
        _stage_a_body,
        grid=(1,),
        in_specs=[
            pl.BlockSpec((B, S), lambda i: (0, 0)),
            pl.BlockSpec((VPAD, EMBED_DIM), lambda i: (0, 0)),
            pl.BlockSpec((1, EMBED_DIM), lambda i: (0, 0)),
            pl.BlockSpec((1, EMBED_DIM), lambda i: (0, 0)),
        ],
        out_specs=pl.BlockSpec((B, 2 * VPAD, EMBED_DIM), lambda i: (0, 0, 0)),
        out_shape=jax.ShapeDtypeStruct((B, 2 * VPAD, EMBED_DIM), jnp.bfloat16),
    )(
        tokens,
        table_pad,
        ln_gamma.reshape(1, EMBED_DIM),
        ln_beta.reshape(1, EMBED_DIM),
    )

    out = pl.pallas_call(
        _stage_b_body,
        grid=(B // RPG,),
        in_specs=[
            pl.BlockSpec((1, 1, TBLK), lambda p: (p, 0, 0)),
            pl.BlockSpec((K, EMBED_DIM), lambda p: (p, 0)),
        ],
        out_specs=pl.BlockSpec((TBLK, EMBED_DIM), lambda p: (p, 0)),
        out_shape=jax.ShapeDtypeStruct((B * S, EMBED_DIM), jnp.float32),
    )(
        tokens.reshape(B // RPG, 1, TBLK),
        n2.reshape(B * 2 * VPAD, EMBED_DIM),
    )
    return out.reshape(B, S, EMBED_DIM)
